# baseline (speedup 1.0000x reference)
import functools

import jax
import jax.numpy as jnp
from jax import lax
from jax.experimental import pallas as pl
from jax.experimental.pallas import tpu as pltpu
from jax.experimental.pallas import tpu_sc as plsc

N_PROT = 10000
N_DRUG = 10000
NUM_ET = 64

NC = 2
NS = 16
NW = NC * NS
SUB = 128

M_PAD = 10240
SINK = M_PAD - 1


def _pad_edges(e, g):
    q = NW * SUB * g * 2
    return -(-e // q) * q


@functools.cache
def _sc_gather_scatter(d, e_pad, n_out, g):
    nsub_w = e_pad // (NW * SUB)
    ngroups = nsub_w // g
    assert ngroups % 2 == 0
    rpt = n_out // NS
    mesh = plsc.VectorSubcoreMesh(core_axis_name="c", subcore_axis_name="s")

    @functools.partial(
        pl.kernel,
        out_type=jax.ShapeDtypeStruct((NC, n_out, d), jnp.float32),
        mesh=mesh,
        scratch_types=[
            pltpu.VMEM((nsub_w + g, SUB), jnp.int32),
            pltpu.VMEM((nsub_w, SUB), jnp.int32),
            pltpu.VMEM((g * SUB, d), jnp.float32),
            pltpu.VMEM((g * SUB, d), jnp.float32),
            pltpu.VMEM((rpt, d), jnp.float32),
            pltpu.VMEM_SHARED((n_out, d), jnp.float32),
            pltpu.SemaphoreType.DMA,
            pltpu.SemaphoreType.DMA,
            pltpu.SemaphoreType.DMA,
        ],
        compiler_params=pltpu.CompilerParams(use_tc_tiling_on_sc=False),
    )
    def k(table, gidx, didx, zeros, out, gbuf, dbuf, rows_a, rows_b, obuf,
          accum, gsem_a, gsem_b, ssem):
        cid = lax.axis_index("c")
        sid = lax.axis_index("s")
        wid = sid * NC + cid

        def fire(buf, sem, base):
            return [pltpu.async_copy(table.at[gbuf.at[base + j]],
                                     buf.at[pl.ds(j * SUB, SUB)], sem)
                    for j in range(g)]

        def drain(buf, sem):
            for j in range(g):
                pltpu.make_async_copy(table.at[gbuf.at[j]],
                                      buf.at[pl.ds(j * SUB, SUB)], sem).wait()

        def scatter(buf, base):
            cps = [pltpu.async_copy(buf.at[pl.ds(j * SUB, SUB)],
                                    accum.at[dbuf.at[base + j]], ssem,
                                    add=True)
                   for j in range(g)]
            for c in cps:
                c.wait()

        pltpu.sync_copy(gidx.at[pl.ds(wid * nsub_w, nsub_w)],
                        gbuf.at[pl.ds(0, nsub_w)])
        pltpu.sync_copy(didx.at[pl.ds(wid * nsub_w, nsub_w)], dbuf)
        for j in range(g):
            for t in range(SUB // 16):
                gbuf[nsub_w + j, pl.ds(t * 16, 16)] = jnp.zeros(
                    (16,), jnp.int32)
        pltpu.sync_copy(zeros.at[pl.ds(sid * rpt, rpt)], obuf)
        pltpu.sync_copy(obuf, accum.at[pl.ds(sid * rpt, rpt)])
        plsc.subcore_barrier()

        fire(rows_a, gsem_a, 0)

        def body(gg, carry):
            s_a = 2 * gg * g
            fire(rows_b, gsem_b, s_a + g)
            drain(rows_a, gsem_a)
            scatter(rows_a, s_a)
            fire(rows_a, gsem_a, s_a + 2 * g)
            drain(rows_b, gsem_b)
            scatter(rows_b, s_a + g)
            return carry

        lax.fori_loop(0, ngroups // 2, body, 0)
        drain(rows_a, gsem_a)
        plsc.subcore_barrier()
        pltpu.sync_copy(accum.at[pl.ds(sid * rpt, rpt)], obuf)
        pltpu.sync_copy(obuf, out.at[cid, pl.ds(sid * rpt, rpt)])

    return k


@functools.cache
def _sc_counts(e_pad, n_out, g=8):
    d = 16
    nsub_w = e_pad // (NW * SUB)
    ngroups = nsub_w // g
    rpt = n_out // NS
    mesh = plsc.VectorSubcoreMesh(core_axis_name="c", subcore_axis_name="s")

    @functools.partial(
        pl.kernel,
        out_type=jax.ShapeDtypeStruct((NC, n_out, d), jnp.float32),
        mesh=mesh,
        scratch_types=[
            pltpu.VMEM((nsub_w, SUB), jnp.int32),
            pltpu.VMEM((SUB, d), jnp.float32),
            pltpu.VMEM((rpt, d), jnp.float32),
            pltpu.VMEM_SHARED((n_out, d), jnp.float32),
            pltpu.SemaphoreType.DMA,
        ],
        compiler_params=pltpu.CompilerParams(use_tc_tiling_on_sc=False),
    )
    def k(ones, didx, zeros, out, dbuf, obuf_ones, obuf, accum, ssem):
        cid = lax.axis_index("c")
        sid = lax.axis_index("s")
        wid = sid * NC + cid
        pltpu.sync_copy(ones, obuf_ones)
        pltpu.sync_copy(didx.at[pl.ds(wid * nsub_w, nsub_w)], dbuf)
        pltpu.sync_copy(zeros.at[pl.ds(sid * rpt, rpt)], obuf)
        pltpu.sync_copy(obuf, accum.at[pl.ds(sid * rpt, rpt)])
        plsc.subcore_barrier()

        def body(gi, carry):
            base = gi * g
            cps = [pltpu.async_copy(obuf_ones, accum.at[dbuf.at[base + j]],
                                    ssem, add=True)
                   for j in range(g)]
            for c in cps:
                c.wait()
            return carry

        lax.fori_loop(0, ngroups, body, 0)
        plsc.subcore_barrier()
        pltpu.sync_copy(accum.at[pl.ds(sid * rpt, rpt)], obuf)
        pltpu.sync_copy(obuf, out.at[cid, pl.ds(sid * rpt, rpt)])

    return k


def _pad1(a, n, fill):
    return jnp.concatenate(
        [a, jnp.full((n - a.shape[0],), fill, a.dtype)])


def _pad_rows(a, n):
    return jnp.concatenate(
        [a, jnp.zeros((n - a.shape[0],) + a.shape[1:], a.dtype)])


BLK = 512
NBLK = M_PAD // BLK


def _dot(a, b):
    return jnp.dot(a, b, preferred_element_type=jnp.float32)


def _tc_weights(att1, b1f, att2, b2f):
    def body(a1, b1, a2, b2, t1, t2):
        t1[...] = _dot(a1[...], b1[...])
        t2[...] = _dot(a2[...], b2[...])
    return pl.pallas_call(
        body,
        out_shape=[
            jax.ShapeDtypeStruct((att1.shape[0], b1f.shape[1]), jnp.float32),
            jax.ShapeDtypeStruct((att2.shape[0], b2f.shape[1]), jnp.float32),
        ],
    )(att1, b1f, att2, b2f)


def _cnt_spec(section):
    return pl.BlockSpec((2, BLK, 16), lambda i, s=section: (0, s * NBLK + i, 0))


def _row_spec(d):
    return pl.BlockSpec((BLK, d), lambda i: (i, 0))


def _full_spec(shape):
    nd = len(shape)
    return pl.BlockSpec(shape, lambda i: (0,) * nd)


def _tc_gcn1(x_prot, w1, cp):
    def body(x, w, c, o):
        dinv = lax.rsqrt(c[0, :, 0:1] + c[1, :, 0:1] + 1.0)
        o[...] = _dot(x[...], w[...]) * dinv
    return pl.pallas_call(
        body, grid=(NBLK,),
        in_specs=[_row_spec(128), _full_spec(w1.shape), _cnt_spec(0)],
        out_specs=_row_spec(32),
        out_shape=jax.ShapeDtypeStruct((M_PAD, 32), jnp.float32),
    )(x_prot, w1, cp)


def _tc_gcn2(s1p, g1, cp, w2, b1):
    def body(s, g, c, w, b, o):
        dinv = lax.rsqrt(c[0, :, 0:1] + c[1, :, 0:1] + 1.0)
        xp1 = jnp.maximum(dinv * (s[0] + s[1] + g[...]) + b[...], 0.0)
        o[...] = _dot(xp1, w[...]) * dinv
    return pl.pallas_call(
        body, grid=(NBLK,),
        in_specs=[pl.BlockSpec((2, BLK, 32), lambda i: (0, i, 0)),
                  _row_spec(32), _cnt_spec(0), _full_spec(w2.shape),
                  _full_spec(b1.shape)],
        out_specs=_row_spec(16),
        out_shape=jax.ShapeDtypeStruct((M_PAD, 16), jnp.float32),
    )(s1p, g1, cp, w2, b1)


def _tc_gcn_out(s2p, g2, cp, b2):
    def body(s, g, c, b, o):
        dinv = lax.rsqrt(c[0, :, 0:1] + c[1, :, 0:1] + 1.0)
        o[...] = dinv * (s[0] + s[1] + g[...]) + b[...]
    return pl.pallas_call(
        body, grid=(NBLK,),
        in_specs=[pl.BlockSpec((2, BLK, 16), lambda i: (0, i, 0)),
                  _row_spec(16), _cnt_spec(0), _full_spec(b2.shape)],
        out_specs=_row_spec(16),
        out_shape=jax.ShapeDtypeStruct((M_PAD, 16), jnp.float32),
    )(s2p, g2, cp, b2)


def _tc_drug1(s3p, cp, hgcn_w, x_drug, embed, d_norm2, w1a, w1b, r1a, r1b):
    def body(s, c, hg, xd, em, dn, wa, wb, ra, rb, y, zr):
        cnt = jnp.maximum(c[0, :, 0:1] + c[1, :, 0:1], 1.0)
        xh = _dot((s[0] + s[1]) / cnt, hg[...])
        x0 = _dot(xd[...], em[...]) / dn[...]
        y[...] = _dot(x0, wa[...]) + _dot(xh, wb[...])
        zr[...] = _dot(x0, ra[...]) + _dot(xh, rb[...])
    return pl.pallas_call(
        body, grid=(NBLK,),
        in_specs=[pl.BlockSpec((2, BLK, 16), lambda i: (0, i, 0)),
                  _cnt_spec(2), _full_spec(hgcn_w.shape), _row_spec(128),
                  _full_spec(embed.shape), pl.BlockSpec((BLK, 1), lambda i: (i, 0)),
                  _full_spec(w1a.shape), _full_spec(w1b.shape),
                  _full_spec(r1a.shape), _full_spec(r1b.shape)],
        out_specs=[_row_spec(w1a.shape[1]), _row_spec(r1a.shape[1])],
        out_shape=[jax.ShapeDtypeStruct((M_PAD, w1a.shape[1]), jnp.float32),
                   jax.ShapeDtypeStruct((M_PAD, r1a.shape[1]), jnp.float32)],
    )(s3p, cp, hgcn_w, x_drug, embed, d_norm2, w1a, w1b, r1a, r1b)


def _tc_drug2(s4p, cp, zr1, w2cat, root2):
    def body(s, c, z, w, r, y, zr):
        cnt = jnp.maximum(c[0, :, 0:1] + c[1, :, 0:1], 1.0)
        xd1 = jnp.maximum((s[0] + s[1]) / cnt + z[...], 0.0)
        y[...] = _dot(xd1, w[...])
        zr[...] = _dot(xd1, r[...])
    return pl.pallas_call(
        body, grid=(NBLK,),
        in_specs=[pl.BlockSpec((2, BLK, 32), lambda i: (0, i, 0)),
                  _cnt_spec(1), _row_spec(32), _full_spec(w2cat.shape),
                  _full_spec(root2.shape)],
        out_specs=[_row_spec(w2cat.shape[1]), _row_spec(root2.shape[1])],
        out_shape=[jax.ShapeDtypeStruct((M_PAD, w2cat.shape[1]), jnp.float32),
                   jax.ShapeDtypeStruct((M_PAD, root2.shape[1]), jnp.float32)],
    )(s4p, cp, zr1, w2cat, root2)


def _tc_final(s5p, cp, zr2):
    def body(s, c, z, o):
        cnt = jnp.maximum(c[0, :, 0:1] + c[1, :, 0:1], 1.0)
        o[...] = (s[0] + s[1]) / cnt + z[...]
    return pl.pallas_call(
        body, grid=(NBLK,),
        in_specs=[pl.BlockSpec((2, BLK, 16), lambda i: (0, i, 0)),
                  _cnt_spec(1), _row_spec(16)],
        out_specs=_row_spec(16),
        out_shape=jax.ShapeDtypeStruct((M_PAD, 16), jnp.float32),
    )(s5p, cp, zr2)


def kernel(x_drug, dd_edge_index, dd_edge_type, dd_range_list, d_norm,
           x_prot, pp_edge_index, dp_edge_index, dp_range_list,
           gcn_w1, gcn_b1, gcn_w2, gcn_b2, embed, hgcn_w,
           basis1, att1, root1, basis2, att2, root2):
    e_pp = pp_edge_index.shape[1]
    e_dd = dd_edge_index.shape[1]
    e_dp = dp_edge_index.shape[1]

    pp_src, pp_dst = pp_edge_index[0], pp_edge_index[1]
    dd_src, dd_dst = dd_edge_index[0], dd_edge_index[1]
    dp_src, dp_dst = dp_edge_index[0], dp_edge_index[1]

    g_pp = 8 if _pad_edges(e_pp, 8) == _pad_edges(e_pp, 4) else 4
    g_dd = 8 if _pad_edges(e_dd, 8) == _pad_edges(e_dd, 4) else 4
    g_dp = 8 if _pad_edges(e_dp, 8) == _pad_edges(e_dp, 4) else 4
    e_pp_pad = _pad_edges(e_pp, g_pp)
    e_dd_pad = _pad_edges(e_dd, g_dd)
    e_dp_pad = _pad_edges(e_dp, g_dp)

    def _rows(a):
        return a.reshape(-1, SUB)

    gidx_pp = _rows(_pad1(pp_src, e_pp_pad, 0))
    didx_pp = _rows(_pad1(pp_dst, e_pp_pad, SINK))
    gidx_dd = _rows(_pad1(dd_src * NUM_ET + dd_edge_type, e_dd_pad, 0))
    didx_dd = _rows(_pad1(dd_dst, e_dd_pad, SINK))
    gidx_dp = _rows(_pad1(dp_src, e_dp_pad, 0))
    didx_dp = _rows(_pad1(dp_dst - N_PROT, e_dp_pad, SINK))

    e_cnt = e_pp + e_dd + e_dp
    e_cnt_pad = _pad_edges(e_cnt, 4)
    didx_cnt = _rows(_pad1(
        jnp.concatenate([pp_dst, dd_dst + M_PAD,
                         dp_dst + (2 * M_PAD - N_PROT)]),
        e_cnt_pad, 3 * M_PAD - 1))

    zeros_c = jnp.zeros((3 * M_PAD, 16), jnp.float32)
    zeros_32 = jnp.zeros((M_PAD, 32), jnp.float32)
    zeros_16 = jnp.zeros((M_PAD, 16), jnp.float32)
    ones_r = jnp.ones((SUB, 16), jnp.float32)

    cp = _sc_counts(e_cnt_pad, 3 * M_PAD)(ones_r, didx_cnt, zeros_c)

    gs32_pp = _sc_gather_scatter(32, e_pp_pad, M_PAD, g_pp)
    gs16_pp = _sc_gather_scatter(16, e_pp_pad, M_PAD, g_pp)
    gs32_dd = _sc_gather_scatter(32, e_dd_pad, M_PAD, g_dd)
    gs16_dd = _sc_gather_scatter(16, e_dd_pad, M_PAD, g_dd)
    gs16_dp = _sc_gather_scatter(16, e_dp_pad, M_PAD, g_dp)

    nb = basis1.shape[0]
    d_in1 = basis1.shape[1]
    t1, t2 = _tc_weights(att1, basis1.reshape(nb, -1),
                         att2, basis2.reshape(nb, -1))
    w1cat = jnp.transpose(t1.reshape(NUM_ET, d_in1, -1),
                          (1, 0, 2)).reshape(d_in1, -1)
    w2cat = jnp.transpose(t2.reshape(NUM_ET, basis2.shape[1], -1),
                          (1, 0, 2)).reshape(basis2.shape[1], -1)
    n_emb = embed.shape[1]
    w1a, w1b = w1cat[:n_emb], w1cat[n_emb:]
    r1a, r1b = root1[:n_emb], root1[n_emb:]

    x_prot_p = _pad_rows(x_prot, M_PAD)
    x_drug_p = _pad_rows(x_drug, M_PAD)
    d_norm_p = _pad1(d_norm, M_PAD, 1.0).reshape(M_PAD, 1)

    g1 = _tc_gcn1(x_prot_p, gcn_w1, cp)
    s1p = gs32_pp(g1, gidx_pp, didx_pp, zeros_32)

    g2 = _tc_gcn2(s1p, g1, cp, gcn_w2, gcn_b1.reshape(1, -1))
    s2p = gs16_pp(g2, gidx_pp, didx_pp, zeros_16)
    xp2 = _tc_gcn_out(s2p, g2, cp, gcn_b2.reshape(1, -1))

    s3p = gs16_dp(xp2, gidx_dp, didx_dp, zeros_16)
    y1, zr1 = _tc_drug1(s3p, cp, hgcn_w, x_drug_p, embed, d_norm_p,
                        w1a, w1b, r1a, r1b)
    s4p = gs32_dd(y1.reshape(M_PAD * NUM_ET, -1), gidx_dd, didx_dd, zeros_32)

    y2, zr2 = _tc_drug2(s4p, cp, zr1, w2cat, root2)
    s5p = gs16_dd(y2.reshape(M_PAD * NUM_ET, -1), gidx_dd, didx_dd, zeros_16)
    out = _tc_final(s5p, cp, zr2)
    return out[:N_DRUG]


# speedup vs baseline: 2.0221x; 2.0221x over previous
import functools

import jax
import jax.numpy as jnp
from jax import lax
from jax.experimental import pallas as pl
from jax.experimental.pallas import tpu as pltpu
from jax.experimental.pallas import tpu_sc as plsc

N_PROT = 10000
N_DRUG = 10000
NUM_ET = 64

NC = 2
NS = 16
NW = NC * NS
SUB = 128

M_PAD = 10240
SINK = M_PAD - 1


def _pad_edges(e, g):
    q = NW * SUB * g * 2
    return -(-e // q) * q


@functools.cache
def _sc_gather_scatter(d, e_pad, n_out, g):
    nsub_w = e_pad // (NW * SUB)
    ngroups = nsub_w // g
    assert ngroups % 2 == 0
    rpt = n_out // NS
    mesh = plsc.VectorSubcoreMesh(core_axis_name="c", subcore_axis_name="s")

    @functools.partial(
        pl.kernel,
        out_type=jax.ShapeDtypeStruct((NC, n_out, d), jnp.float32),
        mesh=mesh,
        scratch_types=[
            pltpu.VMEM((g, SUB), jnp.int32),
            pltpu.VMEM((g, SUB), jnp.int32),
            pltpu.VMEM((g, SUB), jnp.int32),
            pltpu.VMEM((g, SUB), jnp.int32),
            pltpu.VMEM((g * SUB, d), jnp.float32),
            pltpu.VMEM((g * SUB, d), jnp.float32),
            pltpu.VMEM((rpt, d), jnp.float32),
            pltpu.VMEM_SHARED((n_out, d), jnp.float32),
            pltpu.SemaphoreType.DMA,
            pltpu.SemaphoreType.DMA,
            pltpu.SemaphoreType.DMA,
            pltpu.SemaphoreType.DMA,
        ],
        compiler_params=pltpu.CompilerParams(use_tc_tiling_on_sc=False),
    )
    def k(table, gidx, didx, zeros, out, gst_a, gst_b, dst_a,
          dst_b, rows_a, rows_b, obuf, accum, gsem_a, gsem_b, ssem, isem):
        cid = lax.axis_index("c")
        sid = lax.axis_index("s")
        wid = sid * NC + cid
        row0 = wid * nsub_w

        def stage(k_grp, gst, dst):
            return [pltpu.async_copy(
                        gidx.at[pl.ds(row0 + k_grp * g, g)], gst, isem),
                    pltpu.async_copy(
                        didx.at[pl.ds(row0 + k_grp * g, g)], dst, isem)]

        def stage_wait(cps):
            for c in cps:
                c.wait()

        def fire(gst, buf, sem):
            return [pltpu.async_copy(table.at[gst.at[j]],
                                     buf.at[pl.ds(j * SUB, SUB)], sem)
                    for j in range(g)]

        def drain(gst, buf, sem):
            for j in range(g):
                pltpu.make_async_copy(table.at[gst.at[j]],
                                      buf.at[pl.ds(j * SUB, SUB)], sem).wait()

        def scatter(dst, buf):
            cps = [pltpu.async_copy(buf.at[pl.ds(j * SUB, SUB)],
                                    accum.at[dst.at[j]], ssem, add=True)
                   for j in range(g)]
            for c in cps:
                c.wait()

        pltpu.sync_copy(zeros.at[pl.ds(sid * rpt, rpt)], obuf)
        pltpu.sync_copy(obuf, accum.at[pl.ds(sid * rpt, rpt)])
        plsc.subcore_barrier()

        stage_wait(stage(0, gst_a, dst_a))
        fire(gst_a, rows_a, gsem_a)
        stage_wait(stage(1, gst_b, dst_b))
        last = ngroups - 1

        def body(gg, carry):
            k_a = 2 * gg
            fire(gst_b, rows_b, gsem_b)
            drain(gst_a, rows_a, gsem_a)
            scatter(dst_a, rows_a)
            stage_wait(stage(jnp.minimum(k_a + 2, last), gst_a, dst_a))
            fire(gst_a, rows_a, gsem_a)
            drain(gst_b, rows_b, gsem_b)
            scatter(dst_b, rows_b)
            stage_wait(stage(jnp.minimum(k_a + 3, last), gst_b, dst_b))
            return carry

        lax.fori_loop(0, ngroups // 2, body, 0)
        drain(gst_a, rows_a, gsem_a)
        plsc.subcore_barrier()
        pltpu.sync_copy(accum.at[pl.ds(sid * rpt, rpt)], obuf)
        pltpu.sync_copy(obuf, out.at[cid, pl.ds(sid * rpt, rpt)])

    return k


@functools.cache
def _sc_counts(e_pad, n_out, g=4):
    d = 16
    nsub_w = e_pad // (NW * SUB)
    ngroups = nsub_w // g
    assert ngroups % 2 == 0
    rpt = n_out // NS
    mesh = plsc.VectorSubcoreMesh(core_axis_name="c", subcore_axis_name="s")

    @functools.partial(
        pl.kernel,
        out_type=jax.ShapeDtypeStruct((NC, n_out, d), jnp.float32),
        mesh=mesh,
        scratch_types=[
            pltpu.VMEM((g, SUB), jnp.int32),
            pltpu.VMEM((g, SUB), jnp.int32),
            pltpu.VMEM((SUB, d), jnp.float32),
            pltpu.VMEM((rpt, d), jnp.float32),
            pltpu.VMEM_SHARED((n_out, d), jnp.float32),
            pltpu.SemaphoreType.DMA,
            pltpu.SemaphoreType.DMA,
            pltpu.SemaphoreType.DMA,
        ],
        compiler_params=pltpu.CompilerParams(use_tc_tiling_on_sc=False),
    )
    def k(ones, didx, zeros, out, dst_a, dst_b, obuf_ones, obuf, accum,
          ssem, isem_a, isem_b):
        cid = lax.axis_index("c")
        sid = lax.axis_index("s")
        wid = sid * NC + cid
        row0 = wid * nsub_w
        pltpu.sync_copy(ones, obuf_ones)
        pltpu.sync_copy(zeros.at[pl.ds(sid * rpt, rpt)], obuf)
        pltpu.sync_copy(obuf, accum.at[pl.ds(sid * rpt, rpt)])
        plsc.subcore_barrier()

        def stage(k_grp, dst, sem):
            return pltpu.async_copy(
                didx.at[pl.ds(row0 + k_grp * g, g)], dst, sem)

        def stage_wait(dst, sem):
            pltpu.make_async_copy(didx.at[pl.ds(row0, g)], dst, sem).wait()

        def scatter(dst):
            cps = [pltpu.async_copy(obuf_ones, accum.at[dst.at[j]],
                                    ssem, add=True)
                   for j in range(g)]
            for c in cps:
                c.wait()

        stage(0, dst_a, isem_a).wait()
        stage(1, dst_b, isem_b)
        last = ngroups - 1

        def body(gg, carry):
            k_a = 2 * gg
            scatter(dst_a)
            stage(jnp.minimum(k_a + 2, last), dst_a, isem_a)
            stage_wait(dst_b, isem_b)
            scatter(dst_b)
            stage_wait(dst_a, isem_a)
            stage(jnp.minimum(k_a + 3, last), dst_b, isem_b)
            return carry

        lax.fori_loop(0, ngroups // 2, body, 0)
        stage_wait(dst_b, isem_b)
        plsc.subcore_barrier()
        pltpu.sync_copy(accum.at[pl.ds(sid * rpt, rpt)], obuf)
        pltpu.sync_copy(obuf, out.at[cid, pl.ds(sid * rpt, rpt)])

    return k


def _pad1(a, n, fill):
    return jnp.concatenate(
        [a, jnp.full((n - a.shape[0],), fill, a.dtype)])


def _pad_rows(a, n):
    return jnp.concatenate(
        [a, jnp.zeros((n - a.shape[0],) + a.shape[1:], a.dtype)])


BLK = 512
NBLK = M_PAD // BLK


def _dot(a, b):
    return jnp.dot(a, b, preferred_element_type=jnp.float32)


def _tc_weights(att1, b1f, att2, b2f):
    def body(a1, b1, a2, b2, t1, t2):
        t1[...] = _dot(a1[...], b1[...])
        t2[...] = _dot(a2[...], b2[...])
    return pl.pallas_call(
        body,
        out_shape=[
            jax.ShapeDtypeStruct((att1.shape[0], b1f.shape[1]), jnp.float32),
            jax.ShapeDtypeStruct((att2.shape[0], b2f.shape[1]), jnp.float32),
        ],
    )(att1, b1f, att2, b2f)


def _cnt_spec(section):
    return pl.BlockSpec((2, BLK, 16), lambda i, s=section: (0, s * NBLK + i, 0))


def _row_spec(d):
    return pl.BlockSpec((BLK, d), lambda i: (i, 0))


def _full_spec(shape):
    nd = len(shape)
    return pl.BlockSpec(shape, lambda i: (0,) * nd)


def _tc_gcn1(x_prot, w1, cp):
    def body(x, w, c, o):
        dinv = lax.rsqrt(c[0, :, 0:1] + c[1, :, 0:1] + 1.0)
        o[...] = _dot(x[...], w[...]) * dinv
    return pl.pallas_call(
        body, grid=(NBLK,),
        in_specs=[_row_spec(128), _full_spec(w1.shape), _cnt_spec(0)],
        out_specs=_row_spec(32),
        out_shape=jax.ShapeDtypeStruct((M_PAD, 32), jnp.float32),
    )(x_prot, w1, cp)


def _tc_gcn2(s1p, g1, cp, w2, b1):
    def body(s, g, c, w, b, o):
        dinv = lax.rsqrt(c[0, :, 0:1] + c[1, :, 0:1] + 1.0)
        xp1 = jnp.maximum(dinv * (s[0] + s[1] + g[...]) + b[...], 0.0)
        o[...] = _dot(xp1, w[...]) * dinv
    return pl.pallas_call(
        body, grid=(NBLK,),
        in_specs=[pl.BlockSpec((2, BLK, 32), lambda i: (0, i, 0)),
                  _row_spec(32), _cnt_spec(0), _full_spec(w2.shape),
                  _full_spec(b1.shape)],
        out_specs=_row_spec(16),
        out_shape=jax.ShapeDtypeStruct((M_PAD, 16), jnp.float32),
    )(s1p, g1, cp, w2, b1)


def _tc_gcn_out(s2p, g2, cp, b2):
    def body(s, g, c, b, o):
        dinv = lax.rsqrt(c[0, :, 0:1] + c[1, :, 0:1] + 1.0)
        o[...] = dinv * (s[0] + s[1] + g[...]) + b[...]
    return pl.pallas_call(
        body, grid=(NBLK,),
        in_specs=[pl.BlockSpec((2, BLK, 16), lambda i: (0, i, 0)),
                  _row_spec(16), _cnt_spec(0), _full_spec(b2.shape)],
        out_specs=_row_spec(16),
        out_shape=jax.ShapeDtypeStruct((M_PAD, 16), jnp.float32),
    )(s2p, g2, cp, b2)


def _tc_drug1(s3p, cp, hgcn_w, x_drug, embed, d_norm2, w1a, w1b, r1a, r1b):
    def body(s, c, hg, xd, em, dn, wa, wb, ra, rb, y, zr):
        cnt = jnp.maximum(c[0, :, 0:1] + c[1, :, 0:1], 1.0)
        xh = _dot((s[0] + s[1]) / cnt, hg[...])
        x0 = _dot(xd[...], em[...]) / dn[...]
        y[...] = _dot(x0, wa[...]) + _dot(xh, wb[...])
        zr[...] = _dot(x0, ra[...]) + _dot(xh, rb[...])
    return pl.pallas_call(
        body, grid=(NBLK,),
        in_specs=[pl.BlockSpec((2, BLK, 16), lambda i: (0, i, 0)),
                  _cnt_spec(2), _full_spec(hgcn_w.shape), _row_spec(128),
                  _full_spec(embed.shape), pl.BlockSpec((BLK, 1), lambda i: (i, 0)),
                  _full_spec(w1a.shape), _full_spec(w1b.shape),
                  _full_spec(r1a.shape), _full_spec(r1b.shape)],
        out_specs=[_row_spec(w1a.shape[1]), _row_spec(r1a.shape[1])],
        out_shape=[jax.ShapeDtypeStruct((M_PAD, w1a.shape[1]), jnp.float32),
                   jax.ShapeDtypeStruct((M_PAD, r1a.shape[1]), jnp.float32)],
    )(s3p, cp, hgcn_w, x_drug, embed, d_norm2, w1a, w1b, r1a, r1b)


def _tc_drug2(s4p, cp, zr1, w2cat, root2):
    def body(s, c, z, w, r, y, zr):
        cnt = jnp.maximum(c[0, :, 0:1] + c[1, :, 0:1], 1.0)
        xd1 = jnp.maximum((s[0] + s[1]) / cnt + z[...], 0.0)
        y[...] = _dot(xd1, w[...])
        zr[...] = _dot(xd1, r[...])
    return pl.pallas_call(
        body, grid=(NBLK,),
        in_specs=[pl.BlockSpec((2, BLK, 32), lambda i: (0, i, 0)),
                  _cnt_spec(1), _row_spec(32), _full_spec(w2cat.shape),
                  _full_spec(root2.shape)],
        out_specs=[_row_spec(w2cat.shape[1]), _row_spec(root2.shape[1])],
        out_shape=[jax.ShapeDtypeStruct((M_PAD, w2cat.shape[1]), jnp.float32),
                   jax.ShapeDtypeStruct((M_PAD, root2.shape[1]), jnp.float32)],
    )(s4p, cp, zr1, w2cat, root2)


def _tc_final(s5p, cp, zr2):
    def body(s, c, z, o):
        cnt = jnp.maximum(c[0, :, 0:1] + c[1, :, 0:1], 1.0)
        o[...] = (s[0] + s[1]) / cnt + z[...]
    return pl.pallas_call(
        body, grid=(NBLK,),
        in_specs=[pl.BlockSpec((2, BLK, 16), lambda i: (0, i, 0)),
                  _cnt_spec(1), _row_spec(16)],
        out_specs=_row_spec(16),
        out_shape=jax.ShapeDtypeStruct((M_PAD, 16), jnp.float32),
    )(s5p, cp, zr2)


def kernel(x_drug, dd_edge_index, dd_edge_type, dd_range_list, d_norm,
           x_prot, pp_edge_index, dp_edge_index, dp_range_list,
           gcn_w1, gcn_b1, gcn_w2, gcn_b2, embed, hgcn_w,
           basis1, att1, root1, basis2, att2, root2):
    e_pp = pp_edge_index.shape[1]
    e_dd = dd_edge_index.shape[1]
    e_dp = dp_edge_index.shape[1]

    pp_src, pp_dst = pp_edge_index[0], pp_edge_index[1]
    dd_src, dd_dst = dd_edge_index[0], dd_edge_index[1]
    dp_src, dp_dst = dp_edge_index[0], dp_edge_index[1]

    g_pp = 8 if _pad_edges(e_pp, 8) == _pad_edges(e_pp, 4) else 4
    g_dd = 8 if _pad_edges(e_dd, 8) == _pad_edges(e_dd, 4) else 4
    g_dp = 8 if _pad_edges(e_dp, 8) == _pad_edges(e_dp, 4) else 4
    e_pp_pad = _pad_edges(e_pp, g_pp)
    e_dd_pad = _pad_edges(e_dd, g_dd)
    e_dp_pad = _pad_edges(e_dp, g_dp)

    def _rows(a):
        return a.reshape(-1, SUB)

    gidx_pp = _rows(_pad1(pp_src, e_pp_pad, 0))
    didx_pp = _rows(_pad1(pp_dst, e_pp_pad, SINK))
    gidx_dd = _rows(_pad1(dd_src * NUM_ET + dd_edge_type, e_dd_pad, 0))
    didx_dd = _rows(_pad1(dd_dst, e_dd_pad, SINK))
    gidx_dp = _rows(_pad1(dp_src, e_dp_pad, 0))
    didx_dp = _rows(_pad1(dp_dst - N_PROT, e_dp_pad, SINK))

    e_cnt = e_pp + e_dd + e_dp
    e_cnt_pad = _pad_edges(e_cnt, 4)
    didx_cnt = _rows(_pad1(
        jnp.concatenate([pp_dst, dd_dst + M_PAD,
                         dp_dst + (2 * M_PAD - N_PROT)]),
        e_cnt_pad, 3 * M_PAD - 1))

    zeros_c = jnp.zeros((3 * M_PAD, 16), jnp.float32)
    zeros_32 = jnp.zeros((M_PAD, 32), jnp.float32)
    zeros_16 = jnp.zeros((M_PAD, 16), jnp.float32)
    ones_r = jnp.ones((SUB, 16), jnp.float32)

    cp = _sc_counts(e_cnt_pad, 3 * M_PAD)(ones_r, didx_cnt, zeros_c)

    gs32_pp = _sc_gather_scatter(32, e_pp_pad, M_PAD, g_pp)
    gs16_pp = _sc_gather_scatter(16, e_pp_pad, M_PAD, g_pp)
    gs32_dd = _sc_gather_scatter(32, e_dd_pad, M_PAD, g_dd)
    gs16_dd = _sc_gather_scatter(16, e_dd_pad, M_PAD, g_dd)
    gs16_dp = _sc_gather_scatter(16, e_dp_pad, M_PAD, g_dp)

    nb = basis1.shape[0]
    d_in1 = basis1.shape[1]
    t1, t2 = _tc_weights(att1, basis1.reshape(nb, -1),
                         att2, basis2.reshape(nb, -1))
    w1cat = jnp.transpose(t1.reshape(NUM_ET, d_in1, -1),
                          (1, 0, 2)).reshape(d_in1, -1)
    w2cat = jnp.transpose(t2.reshape(NUM_ET, basis2.shape[1], -1),
                          (1, 0, 2)).reshape(basis2.shape[1], -1)
    n_emb = embed.shape[1]
    w1a, w1b = w1cat[:n_emb], w1cat[n_emb:]
    r1a, r1b = root1[:n_emb], root1[n_emb:]

    x_prot_p = _pad_rows(x_prot, M_PAD)
    x_drug_p = _pad_rows(x_drug, M_PAD)
    d_norm_p = _pad1(d_norm, M_PAD, 1.0).reshape(M_PAD, 1)

    g1 = _tc_gcn1(x_prot_p, gcn_w1, cp)
    s1p = gs32_pp(g1, gidx_pp, didx_pp, zeros_32)

    g2 = _tc_gcn2(s1p, g1, cp, gcn_w2, gcn_b1.reshape(1, -1))
    s2p = gs16_pp(g2, gidx_pp, didx_pp, zeros_16)
    xp2 = _tc_gcn_out(s2p, g2, cp, gcn_b2.reshape(1, -1))

    s3p = gs16_dp(xp2, gidx_dp, didx_dp, zeros_16)
    y1, zr1 = _tc_drug1(s3p, cp, hgcn_w, x_drug_p, embed, d_norm_p,
                        w1a, w1b, r1a, r1b)
    s4p = gs32_dd(y1.reshape(M_PAD * NUM_ET, -1), gidx_dd, didx_dd, zeros_32)

    y2, zr2 = _tc_drug2(s4p, cp, zr1, w2cat, root2)
    s5p = gs16_dd(y2.reshape(M_PAD * NUM_ET, -1), gidx_dd, didx_dd, zeros_16)
    out = _tc_final(s5p, cp, zr2)
    return out[:N_DRUG]
